# Initial kernel scaffold; baseline (speedup 1.0000x reference)
import jax
import jax.numpy as jnp
from jax.experimental import pallas as pl


def kernel(exec_x, exec_adj, file_x, file_adj, We1, be1, We2, be2, We3, be3, We4, be4, We5, be5, Wf1, bf1, Wf2, bf2, Wf3, bf3, Wf4, bf4, Wf5, bf5, A1, ab1, a2):
    raise NotImplementedError("write your pallas kernel here")


# speedup vs baseline: 1.1479x; 1.1479x over previous
import jax
import jax.numpy as jnp
from jax.experimental import pallas as pl


def _blk(n, max_r, mult):
    best = None
    for r in range(mult, max_r + 1, mult):
        if n % r == 0:
            best = r
    return best if best is not None else n


def _lin_kernel(x_ref, w_ref, o_ref):
    o_ref[...] = jnp.dot(
        x_ref[...].astype(jnp.bfloat16), w_ref[...],
        preferred_element_type=jnp.float32).astype(jnp.bfloat16)


def _lin(x, w):
    n, din = x.shape
    dout = w.shape[1]
    r = _blk(n, 2000, 16)
    return pl.pallas_call(
        _lin_kernel,
        grid=(n // r,),
        in_specs=[
            pl.BlockSpec((r, din), lambda i: (i, 0)),
            pl.BlockSpec((din, dout), lambda i: (0, 0)),
        ],
        out_specs=pl.BlockSpec((r, dout), lambda i: (i, 0)),
        out_shape=jax.ShapeDtypeStruct((n, dout), jnp.bfloat16),
    )(x, w)


def _layer_first_kernel(adj_ref, y_ref, b_ref, wn_ref, yn_ref, adjb_ref):
    a = adj_ref[...].astype(jnp.bfloat16)
    adjb_ref[...] = a
    z = jnp.dot(a, y_ref[...], preferred_element_type=jnp.float32) + b_ref[...]
    h = jnp.maximum(z, 0.0).astype(jnp.bfloat16)
    yn_ref[...] = jnp.dot(h, wn_ref[...],
                          preferred_element_type=jnp.float32).astype(jnp.bfloat16)


def _layer_first(adj, y, b, wn):
    n = adj.shape[0]
    h = y.shape[1]
    hn = wn.shape[1]
    r = _blk(n, 400, 16)
    return pl.pallas_call(
        _layer_first_kernel,
        grid=(n // r,),
        in_specs=[
            pl.BlockSpec((r, n), lambda i: (i, 0)),
            pl.BlockSpec((n, h), lambda i: (0, 0)),
            pl.BlockSpec((1, h), lambda i: (0, 0)),
            pl.BlockSpec((h, hn), lambda i: (0, 0)),
        ],
        out_specs=[
            pl.BlockSpec((r, hn), lambda i: (i, 0)),
            pl.BlockSpec((r, n), lambda i: (i, 0)),
        ],
        out_shape=[
            jax.ShapeDtypeStruct((n, hn), jnp.bfloat16),
            jax.ShapeDtypeStruct((n, n), jnp.bfloat16),
        ],
    )(adj, y, b, wn)


def _layer_mid_kernel(adjb_ref, y_ref, b_ref, wn_ref, yn_ref):
    z = jnp.dot(adjb_ref[...], y_ref[...],
                preferred_element_type=jnp.float32) + b_ref[...]
    hid = jnp.maximum(z, 0.0).astype(jnp.bfloat16)
    yn_ref[...] = jnp.dot(hid, wn_ref[...],
                          preferred_element_type=jnp.float32).astype(jnp.bfloat16)


def _layer_mid(adjb, y, b, wn):
    n = adjb.shape[0]
    h = y.shape[1]
    hn = wn.shape[1]
    r = _blk(n, 400, 16)
    return pl.pallas_call(
        _layer_mid_kernel,
        grid=(n // r,),
        in_specs=[
            pl.BlockSpec((r, n), lambda i: (i, 0)),
            pl.BlockSpec((n, h), lambda i: (0, 0)),
            pl.BlockSpec((1, h), lambda i: (0, 0)),
            pl.BlockSpec((h, hn), lambda i: (0, 0)),
        ],
        out_specs=pl.BlockSpec((r, hn), lambda i: (i, 0)),
        out_shape=jax.ShapeDtypeStruct((n, hn), jnp.bfloat16),
    )(adjb, y, b, wn)


def _layer_last_kernel_relu(adjb_ref, y_ref, b_ref, o_ref):
    z = jnp.dot(adjb_ref[...], y_ref[...],
                preferred_element_type=jnp.float32) + b_ref[...]
    o_ref[...] = jnp.maximum(z, 0.0)


def _layer_last_kernel_id(adjb_ref, y_ref, b_ref, o_ref):
    o_ref[...] = jnp.dot(adjb_ref[...], y_ref[...],
                         preferred_element_type=jnp.float32) + b_ref[...]


def _layer_last(adjb, y, b, relu):
    n = adjb.shape[0]
    h = y.shape[1]
    r = _blk(n, 400, 16)
    body = _layer_last_kernel_relu if relu else _layer_last_kernel_id
    return pl.pallas_call(
        body,
        grid=(n // r,),
        in_specs=[
            pl.BlockSpec((r, n), lambda i: (i, 0)),
            pl.BlockSpec((n, h), lambda i: (0, 0)),
            pl.BlockSpec((1, h), lambda i: (0, 0)),
        ],
        out_specs=pl.BlockSpec((r, h), lambda i: (i, 0)),
        out_shape=jax.ShapeDtypeStruct((n, h), jnp.float32),
    )(adjb, y, b)


def _comb_kernel(ze_ref, xe_ref, zf_ref, xf_ref, a1a_ref, a1b_ref, ab1_ref,
                 a2_ref, z_ref, w0_ref, w1_ref):
    ze = ze_ref[...]
    zf = zf_ref[...]
    ta = jnp.tanh(
        jnp.dot(ze, a1a_ref[...], preferred_element_type=jnp.float32)
        + jnp.dot(xe_ref[...], a1b_ref[...], preferred_element_type=jnp.float32)
        + ab1_ref[...])
    tb = jnp.tanh(
        jnp.dot(zf, a1a_ref[...], preferred_element_type=jnp.float32)
        + jnp.dot(xf_ref[...], a1b_ref[...], preferred_element_type=jnp.float32)
        + ab1_ref[...])
    wa = jnp.sum(ta * a2_ref[...], axis=1, keepdims=True)
    wb = jnp.sum(tb * a2_ref[...], axis=1, keepdims=True)
    m = jnp.maximum(wa, wb)
    ea = jnp.exp(wa - m)
    eb = jnp.exp(wb - m)
    inv = 1.0 / (ea + eb)
    w0 = ea * inv
    w1 = eb * inv
    z_ref[...] = w0 * ze + w1 * zf
    w0_ref[...] = w0
    w1_ref[...] = w1


def _combiner(z_exec, exec_x, z_file, file_x, a1, ab1, a2):
    n, hz = z_exec.shape
    feat = exec_x.shape[1]
    d = a1.shape[0]
    a1a = a1[:hz, :]
    a1b = a1[hz:, :]
    ab1r = ab1.reshape(1, d)
    a2r = a2.reshape(1, d)
    r = _blk(n, 2000, 16)
    z, w0, w1 = pl.pallas_call(
        _comb_kernel,
        grid=(n // r,),
        in_specs=[
            pl.BlockSpec((r, hz), lambda i: (i, 0)),
            pl.BlockSpec((r, feat), lambda i: (i, 0)),
            pl.BlockSpec((r, hz), lambda i: (i, 0)),
            pl.BlockSpec((r, feat), lambda i: (i, 0)),
            pl.BlockSpec((hz, d), lambda i: (0, 0)),
            pl.BlockSpec((feat, d), lambda i: (0, 0)),
            pl.BlockSpec((1, d), lambda i: (0, 0)),
            pl.BlockSpec((1, d), lambda i: (0, 0)),
        ],
        out_specs=[
            pl.BlockSpec((r, hz), lambda i: (i, 0)),
            pl.BlockSpec((r, 1), lambda i: (i, 0)),
            pl.BlockSpec((r, 1), lambda i: (i, 0)),
        ],
        out_shape=[
            jax.ShapeDtypeStruct((n, hz), jnp.float32),
            jax.ShapeDtypeStruct((n, 1), jnp.float32),
            jax.ShapeDtypeStruct((n, 1), jnp.float32),
        ],
    )(z_exec, exec_x, z_file, file_x, a1a, a1b, ab1r, a2r)
    w = jnp.concatenate([w0, w1], axis=1)
    return z, w


def _branch(x, adj, ws, bs, last_relu):
    bs2 = [b.reshape(1, -1) for b in bs]
    wsb = [w.astype(jnp.bfloat16) for w in ws]
    y = _lin(x, wsb[0])
    y, adjb = _layer_first(adj, y, bs2[0], wsb[1])
    for i in range(1, len(ws) - 1):
        y = _layer_mid(adjb, y, bs2[i], wsb[i + 1])
    return _layer_last(adjb, y, bs2[-1], relu=last_relu)


def kernel(exec_x, exec_adj, file_x, file_adj,
           We1, be1, We2, be2, We3, be3, We4, be4, We5, be5,
           Wf1, bf1, Wf2, bf2, Wf3, bf3, Wf4, bf4, Wf5, bf5,
           A1, ab1, a2):
    z_exec = _branch(exec_x, exec_adj,
                     [We1, We2, We3, We4, We5],
                     [be1, be2, be3, be4, be5], last_relu=False)
    z_file = _branch(file_x, file_adj,
                     [Wf1, Wf2, Wf3, Wf4],
                     [bf1, bf2, bf3, bf4], last_relu=True)
    z, w = _combiner(z_exec, exec_x, z_file, file_x, A1, ab1, a2)
    return (z, w, z_exec, z_file)


# speedup vs baseline: 1.3307x; 1.1592x over previous
import jax
import jax.numpy as jnp
from jax.experimental import pallas as pl


def _blk(n, max_r, mult):
    best = None
    for r in range(mult, max_r + 1, mult):
        if n % r == 0:
            best = r
    return best if best is not None else n


def _lin_kernel(x_ref, w_ref, o_ref):
    o_ref[...] = jnp.dot(
        x_ref[...].astype(jnp.bfloat16), w_ref[...],
        preferred_element_type=jnp.float32).astype(jnp.bfloat16)


def _lin(x, w):
    n, din = x.shape
    dout = w.shape[1]
    r = _blk(n, 2000, 16)
    return pl.pallas_call(
        _lin_kernel,
        grid=(n // r,),
        in_specs=[
            pl.BlockSpec((r, din), lambda i: (i, 0)),
            pl.BlockSpec((din, dout), lambda i: (0, 0)),
        ],
        out_specs=pl.BlockSpec((r, dout), lambda i: (i, 0)),
        out_shape=jax.ShapeDtypeStruct((n, dout), jnp.bfloat16),
    )(x, w)


def _layer_first_kernel(adj_ref, y_ref, b_ref, wn_ref, yn_ref, q_ref, s_ref):
    a = adj_ref[...]
    am = jnp.max(a, axis=1, keepdims=True)
    am = jnp.maximum(am, 1e-30)
    inv = 255.0 / am
    qf = jnp.minimum(jnp.round(a * inv), 255.0)
    q_ref[...] = qf.astype(jnp.uint8)[None]
    s = am * (1.0 / 255.0)
    s_ref[...] = s
    z = s * jnp.dot(qf.astype(jnp.bfloat16), y_ref[...],
                    preferred_element_type=jnp.float32) + b_ref[...]
    h = jnp.maximum(z, 0.0).astype(jnp.bfloat16)
    yn_ref[...] = jnp.dot(h, wn_ref[...],
                          preferred_element_type=jnp.float32).astype(jnp.bfloat16)


def _layer_first(adj, y, b, wn):
    n = adj.shape[0]
    h = y.shape[1]
    hn = wn.shape[1]
    r = _blk(n, 400, 16)
    return pl.pallas_call(
        _layer_first_kernel,
        grid=(n // r,),
        in_specs=[
            pl.BlockSpec((r, n), lambda i: (i, 0)),
            pl.BlockSpec((n, h), lambda i: (0, 0)),
            pl.BlockSpec((1, h), lambda i: (0, 0)),
            pl.BlockSpec((h, hn), lambda i: (0, 0)),
        ],
        out_specs=[
            pl.BlockSpec((r, hn), lambda i: (i, 0)),
            pl.BlockSpec((1, r, n), lambda i: (i, 0, 0)),
            pl.BlockSpec((r, 1), lambda i: (i, 0)),
        ],
        out_shape=[
            jax.ShapeDtypeStruct((n, hn), jnp.bfloat16),
            jax.ShapeDtypeStruct((n // r, r, n), jnp.uint8),
            jax.ShapeDtypeStruct((n, 1), jnp.float32),
        ],
    )(adj, y, b, wn)


def _layer_mid_kernel(q_ref, s_ref, y_ref, b_ref, wn_ref, yn_ref):
    z = s_ref[...] * jnp.dot(q_ref[0].astype(jnp.bfloat16), y_ref[...],
                             preferred_element_type=jnp.float32) + b_ref[...]
    hid = jnp.maximum(z, 0.0).astype(jnp.bfloat16)
    yn_ref[...] = jnp.dot(hid, wn_ref[...],
                          preferred_element_type=jnp.float32).astype(jnp.bfloat16)


def _layer_mid(q, s, y, b, wn):
    g, r, n = q.shape
    h = y.shape[1]
    hn = wn.shape[1]
    return pl.pallas_call(
        _layer_mid_kernel,
        grid=(g,),
        in_specs=[
            pl.BlockSpec((1, r, n), lambda i: (i, 0, 0)),
            pl.BlockSpec((r, 1), lambda i: (i, 0)),
            pl.BlockSpec((n, h), lambda i: (0, 0)),
            pl.BlockSpec((1, h), lambda i: (0, 0)),
            pl.BlockSpec((h, hn), lambda i: (0, 0)),
        ],
        out_specs=pl.BlockSpec((r, hn), lambda i: (i, 0)),
        out_shape=jax.ShapeDtypeStruct((n, hn), jnp.bfloat16),
    )(q, s, y, b, wn)


def _layer_last_kernel_relu(q_ref, s_ref, y_ref, b_ref, o_ref):
    z = s_ref[...] * jnp.dot(q_ref[0].astype(jnp.bfloat16), y_ref[...],
                             preferred_element_type=jnp.float32) + b_ref[...]
    o_ref[...] = jnp.maximum(z, 0.0)


def _layer_last_kernel_id(q_ref, s_ref, y_ref, b_ref, o_ref):
    o_ref[...] = s_ref[...] * jnp.dot(
        q_ref[0].astype(jnp.bfloat16), y_ref[...],
        preferred_element_type=jnp.float32) + b_ref[...]


def _layer_last(q, s, y, b, relu):
    g, r, n = q.shape
    h = y.shape[1]
    body = _layer_last_kernel_relu if relu else _layer_last_kernel_id
    return pl.pallas_call(
        body,
        grid=(g,),
        in_specs=[
            pl.BlockSpec((1, r, n), lambda i: (i, 0, 0)),
            pl.BlockSpec((r, 1), lambda i: (i, 0)),
            pl.BlockSpec((n, h), lambda i: (0, 0)),
            pl.BlockSpec((1, h), lambda i: (0, 0)),
        ],
        out_specs=pl.BlockSpec((r, h), lambda i: (i, 0)),
        out_shape=jax.ShapeDtypeStruct((n, h), jnp.float32),
    )(q, s, y, b)


def _comb_kernel(ze_ref, xe_ref, zf_ref, xf_ref, a1a_ref, a1b_ref, ab1_ref,
                 a2_ref, z_ref, w0_ref, w1_ref):
    ze = ze_ref[...]
    zf = zf_ref[...]
    ta = jnp.tanh(
        jnp.dot(ze, a1a_ref[...], preferred_element_type=jnp.float32)
        + jnp.dot(xe_ref[...], a1b_ref[...], preferred_element_type=jnp.float32)
        + ab1_ref[...])
    tb = jnp.tanh(
        jnp.dot(zf, a1a_ref[...], preferred_element_type=jnp.float32)
        + jnp.dot(xf_ref[...], a1b_ref[...], preferred_element_type=jnp.float32)
        + ab1_ref[...])
    wa = jnp.sum(ta * a2_ref[...], axis=1, keepdims=True)
    wb = jnp.sum(tb * a2_ref[...], axis=1, keepdims=True)
    m = jnp.maximum(wa, wb)
    ea = jnp.exp(wa - m)
    eb = jnp.exp(wb - m)
    inv = 1.0 / (ea + eb)
    w0 = ea * inv
    w1 = eb * inv
    z_ref[...] = w0 * ze + w1 * zf
    w0_ref[...] = w0
    w1_ref[...] = w1


def _combiner(z_exec, exec_x, z_file, file_x, a1, ab1, a2):
    n, hz = z_exec.shape
    feat = exec_x.shape[1]
    d = a1.shape[0]
    a1a = a1[:hz, :]
    a1b = a1[hz:, :]
    ab1r = ab1.reshape(1, d)
    a2r = a2.reshape(1, d)
    r = _blk(n, 2000, 16)
    z, w0, w1 = pl.pallas_call(
        _comb_kernel,
        grid=(n // r,),
        in_specs=[
            pl.BlockSpec((r, hz), lambda i: (i, 0)),
            pl.BlockSpec((r, feat), lambda i: (i, 0)),
            pl.BlockSpec((r, hz), lambda i: (i, 0)),
            pl.BlockSpec((r, feat), lambda i: (i, 0)),
            pl.BlockSpec((hz, d), lambda i: (0, 0)),
            pl.BlockSpec((feat, d), lambda i: (0, 0)),
            pl.BlockSpec((1, d), lambda i: (0, 0)),
            pl.BlockSpec((1, d), lambda i: (0, 0)),
        ],
        out_specs=[
            pl.BlockSpec((r, hz), lambda i: (i, 0)),
            pl.BlockSpec((r, 1), lambda i: (i, 0)),
            pl.BlockSpec((r, 1), lambda i: (i, 0)),
        ],
        out_shape=[
            jax.ShapeDtypeStruct((n, hz), jnp.float32),
            jax.ShapeDtypeStruct((n, 1), jnp.float32),
            jax.ShapeDtypeStruct((n, 1), jnp.float32),
        ],
    )(z_exec, exec_x, z_file, file_x, a1a, a1b, ab1r, a2r)
    w = jnp.concatenate([w0, w1], axis=1)
    return z, w


def _branch(x, adj, ws, bs, last_relu):
    bs2 = [b.reshape(1, -1) for b in bs]
    wsb = [w.astype(jnp.bfloat16) for w in ws]
    y = _lin(x, wsb[0])
    y, q, s = _layer_first(adj, y, bs2[0], wsb[1])
    for i in range(1, len(ws) - 1):
        y = _layer_mid(q, s, y, bs2[i], wsb[i + 1])
    return _layer_last(q, s, y, bs2[-1], relu=last_relu)


def kernel(exec_x, exec_adj, file_x, file_adj,
           We1, be1, We2, be2, We3, be3, We4, be4, We5, be5,
           Wf1, bf1, Wf2, bf2, Wf3, bf3, Wf4, bf4, Wf5, bf5,
           A1, ab1, a2):
    z_exec = _branch(exec_x, exec_adj,
                     [We1, We2, We3, We4, We5],
                     [be1, be2, be3, be4, be5], last_relu=False)
    z_file = _branch(file_x, file_adj,
                     [Wf1, Wf2, Wf3, Wf4],
                     [bf1, bf2, bf3, bf4], last_relu=True)
    z, w = _combiner(z_exec, exec_x, z_file, file_x, A1, ab1, a2)
    return (z, w, z_exec, z_file)
